# baseline (speedup 1.0000x reference)
import functools

import jax
import jax.numpy as jnp
from jax import lax
from jax.experimental import pallas as pl
from jax.experimental.pallas import tpu as pltpu
from jax.experimental.pallas import tpu_sc as plsc

_INFO = plsc.get_sparse_core_info()
_NC, _NS = _INFO.num_cores, _INFO.num_subcores
_NW = _NC * _NS


def _body(n, s, d, x_hbm, y_hbm, idx_hbm, out_hbm, idx_v):
    nb = n // _NW
    wid = lax.axis_index("s") * _NC + lax.axis_index("c")
    base = wid * nb
    pltpu.sync_copy(idx_hbm, idx_v)
    idx = idx_v[...][0]
    pltpu.sync_copy(x_hbm.at[pl.ds(base, nb)], out_hbm.at[pl.ds(base, nb)])
    pltpu.sync_copy(
        y_hbm.at[pl.ds(base, nb)],
        out_hbm.at[pl.ds(base, nb), pl.ds(idx, 1)],
    )


def kernel(x, y, dim, index):
    del dim
    n, s, d = x.shape
    idx_arr = jnp.full((16,), index, dtype=jnp.int32)
    y3 = y.reshape(n, 1, d)
    mesh = plsc.VectorSubcoreMesh(core_axis_name="c", subcore_axis_name="s")
    run = pl.kernel(
        functools.partial(_body, n, s, d),
        jax.ShapeDtypeStruct((n, s, d), x.dtype),
        mesh=mesh,
        scratch_types=[pltpu.VMEM((16,), jnp.int32)],
    )
    return run(x, y3, idx_arr)


# speedup vs baseline: 1.0043x; 1.0043x over previous
import jax
import jax.numpy as jnp
from jax.experimental import pallas as pl
from jax.experimental.pallas import tpu as pltpu

_NCHUNK = 8


def _body(idx_ref, x_ref, y_ref, o_ref, sem, ysem):
    nb = x_ref.shape[0] // _NCHUNK
    for k in range(_NCHUNK):
        pltpu.make_async_copy(
            x_ref.at[pl.ds(k * nb, nb)], o_ref.at[pl.ds(k * nb, nb)], sem
        ).start()
    for k in range(_NCHUNK):
        pltpu.make_async_copy(
            x_ref.at[pl.ds(k * nb, nb)], o_ref.at[pl.ds(k * nb, nb)], sem
        ).wait()
    idx = idx_ref[0]
    ycopy = pltpu.make_async_copy(
        y_ref, o_ref.at[:, pl.ds(idx, 1), :], ysem
    )
    ycopy.start()
    ycopy.wait()


def kernel(x, y, dim, index):
    del dim
    n, s, d = x.shape
    idx = jnp.reshape(jnp.asarray(index, jnp.int32), (1,))
    y3 = y.reshape(n, 1, d)
    grid_spec = pltpu.PrefetchScalarGridSpec(
        num_scalar_prefetch=1,
        grid=(1,),
        in_specs=[
            pl.BlockSpec(memory_space=pltpu.MemorySpace.HBM),
            pl.BlockSpec(memory_space=pltpu.MemorySpace.HBM),
        ],
        out_specs=pl.BlockSpec(memory_space=pltpu.MemorySpace.HBM),
        scratch_shapes=[pltpu.SemaphoreType.DMA, pltpu.SemaphoreType.DMA],
    )
    return pl.pallas_call(
        _body,
        grid_spec=grid_spec,
        out_shape=jax.ShapeDtypeStruct((n, s, d), x.dtype),
    )(idx, x, y3)


# speedup vs baseline: 48.8621x; 48.6512x over previous
import jax
import jax.numpy as jnp
from jax.experimental import pallas as pl
from jax.experimental.pallas import tpu as pltpu

_BB = 128


def _body(idx_ref, x_ref, y_ref, o_ref):
    o_ref[...] = x_ref[...]
    idx = idx_ref[0]
    o_ref[:, pl.ds(idx, 1), :] = y_ref[...][:, None, :]


def kernel(x, y, dim, index):
    del dim
    n, s, d = x.shape
    idx = jnp.reshape(jnp.asarray(index, jnp.int32), (1,))
    grid_spec = pltpu.PrefetchScalarGridSpec(
        num_scalar_prefetch=1,
        grid=(n // _BB,),
        in_specs=[
            pl.BlockSpec((_BB, s, d), lambda i, idx_ref: (i, 0, 0)),
            pl.BlockSpec((_BB, d), lambda i, idx_ref: (i, 0)),
        ],
        out_specs=pl.BlockSpec((_BB, s, d), lambda i, idx_ref: (i, 0, 0)),
    )
    return pl.pallas_call(
        _body,
        grid_spec=grid_spec,
        out_shape=jax.ShapeDtypeStruct((n, s, d), x.dtype),
    )(idx, x, y)


# speedup vs baseline: 49.2246x; 1.0074x over previous
import jax
import jax.numpy as jnp
from jax.experimental import pallas as pl
from jax.experimental.pallas import tpu as pltpu

_BB = 128


def _body(idx_ref, x_ref, y_ref, o_ref):
    o_ref[...] = x_ref[...]
    idx = idx_ref[0]
    o_ref[:, pl.ds(idx, 1), :] = y_ref[...][:, None, :]


def kernel(x, y, dim, index):
    del dim
    n, s, d = x.shape
    idx = jnp.reshape(jnp.asarray(index, jnp.int32), (1,))
    grid_spec = pltpu.PrefetchScalarGridSpec(
        num_scalar_prefetch=1,
        grid=(n // _BB,),
        in_specs=[
            pl.BlockSpec((_BB, s, d), lambda i, idx_ref: (i, 0, 0)),
            pl.BlockSpec((_BB, d), lambda i, idx_ref: (i, 0)),
        ],
        out_specs=pl.BlockSpec((_BB, s, d), lambda i, idx_ref: (i, 0, 0)),
    )
    return pl.pallas_call(
        _body,
        grid_spec=grid_spec,
        out_shape=jax.ShapeDtypeStruct((n, s, d), x.dtype),
        compiler_params=pltpu.CompilerParams(
            dimension_semantics=("parallel",),
        ),
    )(idx, x, y)
